# Initial kernel scaffold; baseline (speedup 1.0000x reference)
import jax
import jax.numpy as jnp
from jax.experimental import pallas as pl


def kernel(t, x, W1, b1, W2, b2, g1, be1, g2, be2, vals, src, tgt):
    raise NotImplementedError("write your pallas kernel here")


# speedup vs baseline: 2.7478x; 2.7478x over previous
import functools

import jax
import jax.numpy as jnp
from jax import lax
from jax.experimental import pallas as pl
from jax.experimental.pallas import tpu as pltpu
from jax.experimental.pallas import tpu_sc as plsc

N = 10000
E = 320000
DIM = 128
GROUPS = 32
GSIZE = DIM // GROUPS
EPS = 1e-5

NC = 2
NS = 16
CHUNK = 80
NCHUNKS = 128
E_PAD = NC * NS * NCHUNKS * CHUNK
N_PAD = 10240
ROWS_PER_TILE = N_PAD // NS

ROW_BLK = 1000
GRID = N // ROW_BLK

_HI = jax.lax.Precision.HIGHEST
_WP = jax.lax.Precision.DEFAULT


def _mm_body(x_ref, w_ref, b_ref, o_ref):
    o_ref[...] = (
        jnp.dot(x_ref[...], w_ref[...], preferred_element_type=jnp.float32,
                precision=_WP)
        + b_ref[...]
    )


def _gn(h, g_ref, be_ref):
    r = lax.broadcasted_iota(jnp.int32, (DIM, DIM), 0) // GSIZE
    c = lax.broadcasted_iota(jnp.int32, (DIM, DIM), 1) // GSIZE
    A = jnp.where(r == c, jnp.float32(1.0 / GSIZE), jnp.float32(0.0))
    m = jnp.dot(h, A, preferred_element_type=jnp.float32, precision=_HI)
    msq = jnp.dot(h * h, A, preferred_element_type=jnp.float32, precision=_HI)
    var = msq - m * m
    hn = (h - m) * lax.rsqrt(var + EPS)
    return hn * g_ref[...] + be_ref[...]


def _gn_body(p0_ref, p1_ref, g_ref, be_ref, o_ref):
    h = jnp.maximum(p0_ref[...] + p1_ref[...], 0.0)
    o_ref[...] = _gn(h, g_ref, be_ref)


_row_spec = pl.BlockSpec((ROW_BLK, DIM), lambda i: (i, 0))
_rowk_spec = pl.BlockSpec((ROW_BLK, DIM + 1), lambda i: (i, 0))
_w_spec = pl.BlockSpec((DIM + 1, DIM), lambda i: (0, 0))
_vec_spec = pl.BlockSpec((1, DIM), lambda i: (0, 0))


def _tc_mm(ttx, w, b):
    return pl.pallas_call(
        _mm_body,
        grid=(GRID,),
        in_specs=[_rowk_spec, _w_spec, _vec_spec],
        out_specs=_row_spec,
        out_shape=jax.ShapeDtypeStruct((N, DIM), jnp.float32),
    )(ttx, w, b)


def _tc_gn(p0, p1, g, be):
    return pl.pallas_call(
        _gn_body,
        grid=(GRID,),
        in_specs=[_row_spec, _row_spec, _vec_spec, _vec_spec],
        out_specs=_row_spec,
        out_shape=jax.ShapeDtypeStruct((N, DIM), jnp.float32),
    )(p0, p1, g, be)


def _sc_body(sup, src1, tgt1, val1, out, acc, rows, srcbuf, tgtbuf,
             valbuf, sem):
    c = lax.axis_index("c")
    s = lax.axis_index("s")
    wid = c * NS + s
    zero16 = jnp.zeros((16,), jnp.float32)

    def zfill(r, carry):
        for j in range(DIM // 16):
            rows[r, pl.ds(16 * j, 16)] = zero16
        return carry
    lax.fori_loop(0, CHUNK, zfill, 0)
    for k in range(ROWS_PER_TILE // CHUNK):
        pltpu.sync_copy(rows, acc.at[pl.ds(s * ROWS_PER_TILE + k * CHUNK,
                                           CHUNK)])
    plsc.subcore_barrier()

    def chunk(g, carry):
        base = wid * (NCHUNKS * CHUNK) + g * CHUNK
        d1 = pltpu.async_copy(src1.at[pl.ds(base, CHUNK)], srcbuf, sem)
        d2 = pltpu.async_copy(tgt1.at[pl.ds(base, CHUNK)], tgtbuf, sem)
        d3 = pltpu.async_copy(val1.at[pl.ds(base, CHUNK)], valbuf, sem)
        d1.wait()
        d2.wait()
        d3.wait()

        pltpu.async_copy(sup.at[srcbuf], rows, sem).wait()

        def mgrp(k, c2):
            vvec = valbuf[pl.ds(16 * k, 16)]
            for l in range(16):
                e = 16 * k + l
                vv = jnp.full((16,), vvec[l], jnp.float32)
                for j in range(DIM // 16):
                    sl = pl.ds(16 * j, 16)
                    rows[e, sl] = rows[e, sl] * vv
            return c2
        lax.fori_loop(0, CHUNK // 16, mgrp, 0)

        pltpu.sync_copy(rows, acc.at[tgtbuf], add=True)
        return carry
    lax.fori_loop(0, NCHUNKS, chunk, 0)

    plsc.subcore_barrier()

    row_sl = pl.ds(s * ROWS_PER_TILE, ROWS_PER_TILE)
    pltpu.sync_copy(acc.at[row_sl], out.at[c, row_sl])


@functools.cache
def _make_sc_conv():
    return pl.kernel(
        _sc_body,
        out_type=jax.ShapeDtypeStruct((NC, N_PAD, DIM), jnp.float32),
        mesh=plsc.VectorSubcoreMesh(core_axis_name="c", subcore_axis_name="s",
                                    num_cores=NC, num_subcores=NS),
        scratch_types=[
            pltpu.VMEM_SHARED((N_PAD, DIM), jnp.float32),
            pltpu.VMEM((CHUNK, DIM), jnp.float32),
            pltpu.VMEM((CHUNK,), jnp.int32),
            pltpu.VMEM((CHUNK,), jnp.int32),
            pltpu.VMEM((CHUNK,), jnp.float32),
            pltpu.SemaphoreType.DMA,
        ],
    )


def _sc_conv(sup, src2, tgt2, val2):
    return _make_sc_conv()(sup, src2, tgt2, val2)


def kernel(t, x, W1, b1, W2, b2, g1, be1, g2, be2, vals, src, tgt):
    b1r, b2r = b1[None, :], b2[None, :]
    g1r, be1r = g1[None, :], be1[None, :]
    g2r, be2r = g2[None, :], be2[None, :]
    pad = E_PAD - E
    src1 = jnp.pad(src, (0, pad))
    tgt1 = jnp.pad(tgt, (0, pad))
    val1 = jnp.pad(vals, (0, pad))
    tt = jnp.full((N, 1), t[0], jnp.float32)

    s1 = _tc_mm(jnp.concatenate([tt, x], axis=1), W1, b1r)
    p = _sc_conv(s1, src1, tgt1, val1)
    h1 = _tc_gn(p[0, :N], p[1, :N], g1r, be1r)
    s2 = _tc_mm(jnp.concatenate([tt, h1], axis=1), W2, b2r)
    q = _sc_conv(s2, src1, tgt1, val1)
    return _tc_gn(q[0, :N], q[1, :N], g2r, be2r)


# speedup vs baseline: 8.6994x; 3.1659x over previous
import functools

import jax
import jax.numpy as jnp
from jax import lax
from jax.experimental import pallas as pl
from jax.experimental.pallas import tpu as pltpu
from jax.experimental.pallas import tpu_sc as plsc

N = 10000
E = 320000
DIM = 128
GROUPS = 32
GSIZE = DIM // GROUPS
EPS = 1e-5

NC = 2
NS = 16
CHUNK = 80
EDGES_PER_TILE = E // (NC * NS)
NCHUNKS = EDGES_PER_TILE // CHUNK
N_PAD = 10240
ROWS_PER_TILE = N_PAD // NS

ROW_BLK = 1000
GRID = N // ROW_BLK

_HI = jax.lax.Precision.HIGHEST
_WP = jax.lax.Precision.DEFAULT


def _mm_body(t_ref, x_ref, w_ref, b_ref, o_ref):
    tcol = jnp.full((ROW_BLK, 1), t_ref[0, 0], jnp.float32)
    xt = jnp.concatenate([tcol, x_ref[...]], axis=1)
    o_ref[...] = (
        jnp.dot(xt, w_ref[...], preferred_element_type=jnp.float32,
                precision=_WP)
        + b_ref[...]
    )


def _gn(h, g_ref, be_ref):
    r = lax.broadcasted_iota(jnp.int32, (DIM, DIM), 0) // GSIZE
    c = lax.broadcasted_iota(jnp.int32, (DIM, DIM), 1) // GSIZE
    A = jnp.where(r == c, jnp.float32(1.0 / GSIZE), jnp.float32(0.0))
    m = jnp.dot(h, A, preferred_element_type=jnp.float32, precision=_HI)
    msq = jnp.dot(h * h, A, preferred_element_type=jnp.float32, precision=_HI)
    var = msq - m * m
    hn = (h - m) * lax.rsqrt(var + EPS)
    return hn * g_ref[...] + be_ref[...]


def _gn_body(pa_ref, pb_ref, g_ref, be_ref, o_ref):
    h = jnp.maximum(pa_ref[0] + pb_ref[0], 0.0)
    o_ref[...] = _gn(h, g_ref, be_ref)


_row_spec = pl.BlockSpec((ROW_BLK, DIM), lambda i: (i, 0))
_w_spec = pl.BlockSpec((DIM + 1, DIM), lambda i: (0, 0))
_vec_spec = pl.BlockSpec((1, DIM), lambda i: (0, 0))
_smem_spec = pl.BlockSpec(memory_space=pltpu.SMEM)
_pa_spec = pl.BlockSpec((1, ROW_BLK, DIM), lambda i: (0, i, 0))
_pb_spec = pl.BlockSpec((1, ROW_BLK, DIM), lambda i: (1, i, 0))


def _tc_mm(t11, x, w, b):
    return pl.pallas_call(
        _mm_body,
        grid=(GRID,),
        in_specs=[_smem_spec, _row_spec, _w_spec, _vec_spec],
        out_specs=_row_spec,
        out_shape=jax.ShapeDtypeStruct((N, DIM), jnp.float32),
    )(t11, x, w, b)


def _tc_gn(p, g, be):
    return pl.pallas_call(
        _gn_body,
        grid=(GRID,),
        in_specs=[_pa_spec, _pb_spec, _vec_spec, _vec_spec],
        out_specs=_row_spec,
        out_shape=jax.ShapeDtypeStruct((N, DIM), jnp.float32),
    )(p, p, g, be)


def _sc_body(sup, src1, tgt1, val1, out, acc, rows2, srcb, tgtb, valb,
             semg, semi0, semi1):
    c = lax.axis_index("c")
    s = lax.axis_index("s")
    wid = c * NS + s
    ebase = wid * EDGES_PER_TILE
    zero16 = jnp.zeros((16,), jnp.float32)

    def zfill(r, carry):
        for j in range(DIM // 16):
            rows2[0, r, pl.ds(16 * j, 16)] = zero16
        return carry
    lax.fori_loop(0, CHUNK, zfill, 0)
    for k in range(ROWS_PER_TILE // CHUNK):
        pltpu.sync_copy(rows2.at[0],
                        acc.at[pl.ds(s * ROWS_PER_TILE + k * CHUNK, CHUNK)])
    plsc.subcore_barrier()

    def idx_start(g, b, semi):
        base = ebase + g * CHUNK
        pltpu.async_copy(src1.at[pl.ds(base, CHUNK)], srcb.at[b], semi)
        pltpu.async_copy(tgt1.at[pl.ds(base, CHUNK)], tgtb.at[b], semi)
        pltpu.async_copy(val1.at[pl.ds(base, CHUNK)], valb.at[b], semi)

    def idx_wait(b, semi):
        pltpu.make_async_copy(src1.at[pl.ds(0, CHUNK)], srcb.at[b],
                              semi).wait()
        pltpu.make_async_copy(tgt1.at[pl.ds(0, CHUNK)], tgtb.at[b],
                              semi).wait()
        pltpu.make_async_copy(val1.at[pl.ds(0, CHUNK)], valb.at[b],
                              semi).wait()

    def gather_start(b):
        pltpu.async_copy(sup.at[srcb.at[b]], rows2.at[b], semg)

    def gather_wait(b):
        pltpu.make_async_copy(sup.at[srcb.at[b]], rows2.at[b], semg).wait()

    def scale_scatter(b):
        def mgrp(k, c2):
            vvec = valb[b, pl.ds(16 * k, 16)]
            for l in range(16):
                e = 16 * k + l
                vv = jnp.full((16,), vvec[l], jnp.float32)
                for j in range(DIM // 16):
                    sl = pl.ds(16 * j, 16)
                    rows2[b, e, sl] = rows2[b, e, sl] * vv
            return c2
        lax.fori_loop(0, CHUNK // 16, mgrp, 0)
        pltpu.sync_copy(rows2.at[b], acc.at[tgtb.at[b]], add=True)

    def step(g, b, bn, semi_b, semi_bn):
        gather_wait(b)

        @pl.when(g + 1 < NCHUNKS)
        def _():
            idx_wait(bn, semi_bn)
            gather_start(bn)
        scale_scatter(b)

        @pl.when(g + 2 < NCHUNKS)
        def _():
            idx_start(g + 2, b, semi_b)

    idx_start(0, 0, semi0)
    idx_start(1, 1, semi1)
    idx_wait(0, semi0)
    gather_start(0)

    def pair(i, carry):
        step(2 * i, 0, 1, semi0, semi1)
        step(2 * i + 1, 1, 0, semi1, semi0)
        return carry
    lax.fori_loop(0, NCHUNKS // 2, pair, 0)
    if NCHUNKS % 2:
        step(NCHUNKS - 1, 0, 1, semi0, semi1)

    plsc.subcore_barrier()

    row_sl = pl.ds(s * ROWS_PER_TILE, ROWS_PER_TILE)
    pltpu.sync_copy(acc.at[row_sl], out.at[c, row_sl])


@functools.cache
def _make_sc_conv():
    return pl.kernel(
        _sc_body,
        out_type=jax.ShapeDtypeStruct((NC, N_PAD, DIM), jnp.float32),
        mesh=plsc.VectorSubcoreMesh(core_axis_name="c", subcore_axis_name="s",
                                    num_cores=NC, num_subcores=NS),
        scratch_types=[
            pltpu.VMEM_SHARED((N_PAD, DIM), jnp.float32),
            pltpu.VMEM((2, CHUNK, DIM), jnp.float32),
            pltpu.VMEM((2, CHUNK), jnp.int32),
            pltpu.VMEM((2, CHUNK), jnp.int32),
            pltpu.VMEM((2, CHUNK), jnp.float32),
            pltpu.SemaphoreType.DMA,
            pltpu.SemaphoreType.DMA,
            pltpu.SemaphoreType.DMA,
        ],
    )


def _sc_conv(sup, src1, tgt1, val1):
    return _make_sc_conv()(sup, src1, tgt1, val1)


def kernel(t, x, W1, b1, W2, b2, g1, be1, g2, be2, vals, src, tgt):
    t11 = t.reshape(1, 1)
    b1r, b2r = b1[None, :], b2[None, :]
    g1r, be1r = g1[None, :], be1[None, :]
    g2r, be2r = g2[None, :], be2[None, :]

    s1 = _tc_mm(t11, x, W1, b1r)
    p = _sc_conv(s1, src, tgt, vals)
    h1 = _tc_gn(p, g1r, be1r)
    s2 = _tc_mm(t11, h1, W2, b2r)
    q = _sc_conv(s2, src, tgt, vals)
    return _tc_gn(q, g2r, be2r)


# speedup vs baseline: 9.8171x; 1.1285x over previous
import functools

import jax
import jax.numpy as jnp
from jax import lax
from jax.experimental import pallas as pl
from jax.experimental.pallas import tpu as pltpu
from jax.experimental.pallas import tpu_sc as plsc

N = 10000
E = 320000
DIM = 128
GROUPS = 32
GSIZE = DIM // GROUPS
EPS = 1e-5

NC = 2
NS = 16
CHUNK = 80
EDGES_PER_TILE = E // (NC * NS)
NCHUNKS = EDGES_PER_TILE // CHUNK
N_PAD = 10240
ROWS_PER_TILE = N_PAD // NS

ROW_BLK = 1000
GRID = N // ROW_BLK

_HI = jax.lax.Precision.HIGHEST
_WP = jax.lax.Precision.DEFAULT


def _mm_body(t_ref, x_ref, w_ref, b_ref, o_ref):
    tcol = jnp.full((ROW_BLK, 1), t_ref[0, 0], jnp.float32)
    xt = jnp.concatenate([tcol, x_ref[...]], axis=1)
    o_ref[...] = (
        jnp.dot(xt, w_ref[...], preferred_element_type=jnp.float32,
                precision=_WP)
        + b_ref[...]
    )


def _gn(h, g_ref, be_ref):
    r = lax.broadcasted_iota(jnp.int32, (DIM, DIM), 0) // GSIZE
    c = lax.broadcasted_iota(jnp.int32, (DIM, DIM), 1) // GSIZE
    A = jnp.where(r == c, jnp.float32(1.0 / GSIZE), jnp.float32(0.0))
    m = jnp.dot(h, A, preferred_element_type=jnp.float32, precision=_HI)
    msq = jnp.dot(h * h, A, preferred_element_type=jnp.float32, precision=_HI)
    var = msq - m * m
    hn = (h - m) * lax.rsqrt(var + EPS)
    return hn * g_ref[...] + be_ref[...]


def _gn_body(pa_ref, pb_ref, g_ref, be_ref, o_ref):
    h = jnp.maximum(pa_ref[0] + pb_ref[0], 0.0)
    o_ref[...] = _gn(h, g_ref, be_ref)


def _gnmm_body(t_ref, pa_ref, pb_ref, g_ref, be_ref, w_ref, b_ref, o_ref):
    h = jnp.maximum(pa_ref[0] + pb_ref[0], 0.0)
    hg = _gn(h, g_ref, be_ref)
    tcol = jnp.full((ROW_BLK, 1), t_ref[0, 0], jnp.float32)
    xt = jnp.concatenate([tcol, hg], axis=1)
    o_ref[...] = (
        jnp.dot(xt, w_ref[...], preferred_element_type=jnp.float32,
                precision=_WP)
        + b_ref[...]
    )


_row_spec = pl.BlockSpec((ROW_BLK, DIM), lambda i: (i, 0))
_w_spec = pl.BlockSpec((DIM + 1, DIM), lambda i: (0, 0))
_vec_spec = pl.BlockSpec((1, DIM), lambda i: (0, 0))
_smem_spec = pl.BlockSpec(memory_space=pltpu.SMEM)
_pa_spec = pl.BlockSpec((1, ROW_BLK, DIM), lambda i: (0, i, 0))
_pb_spec = pl.BlockSpec((1, ROW_BLK, DIM), lambda i: (1, i, 0))


def _tc_mm(t11, x, w, b):
    return pl.pallas_call(
        _mm_body,
        grid=(GRID,),
        in_specs=[_smem_spec, _row_spec, _w_spec, _vec_spec],
        out_specs=_row_spec,
        out_shape=jax.ShapeDtypeStruct((N, DIM), jnp.float32),
    )(t11, x, w, b)


def _tc_gn(p, g, be):
    return pl.pallas_call(
        _gn_body,
        grid=(GRID,),
        in_specs=[_pa_spec, _pb_spec, _vec_spec, _vec_spec],
        out_specs=_row_spec,
        out_shape=jax.ShapeDtypeStruct((N, DIM), jnp.float32),
    )(p, p, g, be)


def _tc_gnmm(t11, p, g, be, w, b):
    return pl.pallas_call(
        _gnmm_body,
        grid=(GRID,),
        in_specs=[_smem_spec, _pa_spec, _pb_spec, _vec_spec, _vec_spec,
                  _w_spec, _vec_spec],
        out_specs=_row_spec,
        out_shape=jax.ShapeDtypeStruct((N, DIM), jnp.float32),
    )(t11, p, p, g, be, w, b)


def _sc_body(sup, src1, tgt1, val1, out, acc, rows3, srcb, tgtb, valb,
             semg, semi0, semi1, semi2, sems0, sems1, sems2):
    c = lax.axis_index("c")
    s = lax.axis_index("s")
    wid = c * NS + s
    ebase = wid * EDGES_PER_TILE
    zero16 = jnp.zeros((16,), jnp.float32)
    semi = (semi0, semi1, semi2)
    sems = (sems0, sems1, sems2)

    def zfill(r, carry):
        for j in range(DIM // 16):
            rows3[0, r, pl.ds(16 * j, 16)] = zero16
        return carry
    lax.fori_loop(0, CHUNK, zfill, 0)
    for k in range(ROWS_PER_TILE // CHUNK):
        pltpu.sync_copy(rows3.at[0],
                        acc.at[pl.ds(s * ROWS_PER_TILE + k * CHUNK, CHUNK)])
    plsc.subcore_barrier()

    def idx_start(g, b):
        base = ebase + g * CHUNK
        pltpu.async_copy(src1.at[pl.ds(base, CHUNK)], srcb.at[b], semi[b])
        pltpu.async_copy(tgt1.at[pl.ds(base, CHUNK)], tgtb.at[b], semi[b])
        pltpu.async_copy(val1.at[pl.ds(base, CHUNK)], valb.at[b], semi[b])

    def idx_wait(b):
        z = pl.ds(0, CHUNK)
        pltpu.make_async_copy(src1.at[z], srcb.at[b], semi[b]).wait()
        pltpu.make_async_copy(tgt1.at[z], tgtb.at[b], semi[b]).wait()
        pltpu.make_async_copy(val1.at[z], valb.at[b], semi[b]).wait()

    def gather_start(b):
        pltpu.async_copy(sup.at[srcb.at[b]], rows3.at[b], semg)

    def gather_wait(b):
        pltpu.make_async_copy(sup.at[srcb.at[b]], rows3.at[b], semg).wait()

    def scatter_start(b):
        pltpu.async_copy(rows3.at[b], acc.at[tgtb.at[b]], sems[b], add=True)

    def scatter_wait(b):
        pltpu.make_async_copy(rows3.at[b], acc.at[tgtb.at[b]],
                              sems[b]).wait()

    def scale(b):
        def mgrp(k, c2):
            vvec = valb[b, pl.ds(16 * k, 16)]
            for l in range(16):
                e = 16 * k + l
                vv = jnp.full((16,), vvec[l], jnp.float32)
                for j in range(DIM // 16):
                    sl = pl.ds(16 * j, 16)
                    rows3[b, e, sl] = rows3[b, e, sl] * vv
            return c2
        lax.fori_loop(0, CHUNK // 16, mgrp, 0)

    def step(g, b, wait_prev_scatter=True, gather_next=True, idx_next=True):
        bn, bp = (b + 1) % 3, (b + 2) % 3
        gather_wait(b)
        if gather_next:
            idx_wait(bn)
            gather_start(bn)
        scale(b)
        scatter_start(b)
        if wait_prev_scatter:
            scatter_wait(bp)
        if idx_next:
            idx_start(g + 2, bp)

    idx_start(0, 0)
    idx_start(1, 1)
    idx_wait(0)
    gather_start(0)

    step(0, 0, wait_prev_scatter=False)
    step(1, 1)
    def trio(i, carry):
        g = 2 + 3 * i
        step(g, 2)
        step(g + 1, 0)
        step(g + 2, 1)
        return carry
    lax.fori_loop(0, 40, trio, 0)
    step(122, 2)
    step(123, 0, idx_next=False)
    step(124, 1, gather_next=False, idx_next=False)
    scatter_wait(1)

    plsc.subcore_barrier()

    row_sl = pl.ds(s * ROWS_PER_TILE, ROWS_PER_TILE)
    pltpu.sync_copy(acc.at[row_sl], out.at[c, row_sl])


@functools.cache
def _make_sc_conv():
    return pl.kernel(
        _sc_body,
        out_type=jax.ShapeDtypeStruct((NC, N_PAD, DIM), jnp.float32),
        mesh=plsc.VectorSubcoreMesh(core_axis_name="c", subcore_axis_name="s",
                                    num_cores=NC, num_subcores=NS),
        scratch_types=[
            pltpu.VMEM_SHARED((N_PAD, DIM), jnp.float32),
            pltpu.VMEM((3, CHUNK, DIM), jnp.float32),
            pltpu.VMEM((3, CHUNK), jnp.int32),
            pltpu.VMEM((3, CHUNK), jnp.int32),
            pltpu.VMEM((3, CHUNK), jnp.float32),
            pltpu.SemaphoreType.DMA,
            pltpu.SemaphoreType.DMA,
            pltpu.SemaphoreType.DMA,
            pltpu.SemaphoreType.DMA,
            pltpu.SemaphoreType.DMA,
            pltpu.SemaphoreType.DMA,
            pltpu.SemaphoreType.DMA,
        ],
    )


def _sc_conv(sup, src1, tgt1, val1):
    return _make_sc_conv()(sup, src1, tgt1, val1)


def kernel(t, x, W1, b1, W2, b2, g1, be1, g2, be2, vals, src, tgt):
    t11 = t.reshape(1, 1)
    b1r, b2r = b1[None, :], b2[None, :]
    g1r, be1r = g1[None, :], be1[None, :]
    g2r, be2r = g2[None, :], be2[None, :]

    s1 = _tc_mm(t11, x, W1, b1r)
    p = _sc_conv(s1, src, tgt, vals)
    s2 = _tc_gnmm(t11, p, g1r, be1r, W2, b2r)
    q = _sc_conv(s2, src, tgt, vals)
    return _tc_gn(q, g2r, be2r)
